# baseline (speedup 1.0000x reference)
import math

import jax
import jax.numpy as jnp
from jax.experimental import pallas as pl
from jax.experimental.pallas import tpu as pltpu

_N, _F, _H, _DK, _O = 4096, 128, 4, 32, 128
_ALPHA = 0.2
_SCALE = 1.0 / math.sqrt(_DK)
_HD = _H * _DK
_BM = 512


def _fused_kernel(x_ref, wk_ref, wv_ref, a_ref, g_ref, b_ref, mask8_ref,
                  ow_ref, ob_ref, out_ref, p8_ref, s_ref):
    i = pl.program_id(0)

    @pl.when(i == 0)
    def _prep():
        x = x_ref[...]
        mu = jnp.mean(x, axis=1, keepdims=True)
        xc = x - mu
        var = jnp.mean(xc * xc, axis=1, keepdims=True)
        h = xc * jax.lax.rsqrt(var + 1e-5) * g_ref[...] + b_ref[...]
        k = jnp.dot(h, wk_ref[...], preferred_element_type=jnp.float32)
        v = jnp.dot(h, wv_ref[...], preferred_element_type=jnp.float32)
        lk = jnp.where(k >= 0, k, _ALPHA * k)
        ska = lk * a_ref[...]
        r = jax.lax.broadcasted_iota(jnp.int32, (_HD, _HD), 0) // _DK
        c = jax.lax.broadcasted_iota(jnp.int32, (_HD, _HD), 1) // _DK
        sel = (r == c).astype(jnp.float32)
        ske = jnp.dot(ska, sel, preferred_element_type=jnp.float32)
        m = jnp.max(ske, axis=0, keepdims=True)
        w = jnp.exp((ske - m) * _SCALE)
        wv = w * v
        swv = 127.0 / jnp.max(jnp.abs(wv))
        qn = jnp.clip(jnp.round(wv * swv), -127.0, 127.0).astype(jnp.int8)
        qd = jnp.clip(jnp.round(w * 127.0), 0.0, 127.0).astype(jnp.int8)
        q = jnp.concatenate([qn, qd], axis=1)
        p8_ref[...] = jnp.broadcast_to(
            q[:, None, :], (_N, 4, 2 * _HD)).reshape(4 * _N, 2 * _HD)
        s_ref[...] = jnp.broadcast_to(127.0 / swv, (1, _HD))

    @pl.when(i > 0)
    def _agg():
        agg = jnp.dot(mask8_ref[...], p8_ref[...],
                      preferred_element_type=jnp.int32)
        num = agg[:, :_HD].astype(jnp.float32)
        den = jnp.maximum(agg[:, _HD:], 1).astype(jnp.float32)
        attn = num / den * s_ref[...]
        xblk = x_ref[pl.ds((i - 1) * _BM, _BM), :]
        out_ref[...] = (
            jnp.dot(attn, ow_ref[...], preferred_element_type=jnp.float32)
            + ob_ref[...]
            + xblk
        )


def kernel(x, adj_matrix, W_q, W_k, W_v, a, out_W, out_b, ln_gamma, ln_beta):
    del W_q
    wk2 = W_k.transpose(1, 0, 2).reshape(_F, _HD)
    wv2 = W_v.transpose(1, 0, 2).reshape(_F, _HD)
    a2 = a.reshape(1, _HD)
    g2 = ln_gamma.reshape(1, _F)
    b2 = ln_beta.reshape(1, _F)
    ob2 = out_b.reshape(1, _O)
    adj8 = jax.lax.bitcast_convert_type(adj_matrix, jnp.int8).reshape(_N, 4 * _N)

    out = pl.pallas_call(
        _fused_kernel,
        grid=(1 + _N // _BM,),
        in_specs=[
            pl.BlockSpec((_N, _F), lambda i: (0, 0)),
            pl.BlockSpec((_F, _HD), lambda i: (0, 0)),
            pl.BlockSpec((_F, _HD), lambda i: (0, 0)),
            pl.BlockSpec((1, _HD), lambda i: (0, 0)),
            pl.BlockSpec((1, _F), lambda i: (0, 0)),
            pl.BlockSpec((1, _F), lambda i: (0, 0)),
            pl.BlockSpec((_BM, 4 * _N),
                         lambda i: (jnp.maximum(i - 1, 0), 0)),
            pl.BlockSpec((_F, _O), lambda i: (0, 0)),
            pl.BlockSpec((1, _O), lambda i: (0, 0)),
        ],
        out_specs=pl.BlockSpec((_BM, _O), lambda i: (jnp.maximum(i - 1, 0), 0)),
        out_shape=jax.ShapeDtypeStruct((_N, _O), jnp.float32),
        scratch_shapes=[pltpu.VMEM((4 * _N, 2 * _HD), jnp.int8),
                        pltpu.VMEM((1, _HD), jnp.float32)],
        compiler_params=pltpu.CompilerParams(
            dimension_semantics=("arbitrary",),
        ),
    )(x, wk2, wv2, a2, g2, b2, adj8, out_W, ob2)
    return out


# speedup vs baseline: 24.4090x; 24.4090x over previous
import math

import jax
import jax.numpy as jnp
from jax.experimental import pallas as pl
from jax.experimental.pallas import tpu as pltpu

_N, _F, _H, _DK, _O = 4096, 128, 4, 32, 128
_ALPHA = 0.2
_SCALE = 1.0 / math.sqrt(_DK)
_HD = _H * _DK
_BM = 512


def _agg_body(m_ref, i, p_ref, x_ref, ow_ref, ob_ref, out_ref):
    maskf = (m_ref[...] > 0).astype(jnp.bfloat16)
    agg = jnp.dot(maskf, p_ref[...], preferred_element_type=jnp.float32)
    attn = agg[:, :_HD] / agg[:, _HD:]
    xblk = x_ref[pl.ds((i - 1) * _BM, _BM), :]
    out_ref[...] = (
        jnp.dot(attn, ow_ref[...], preferred_element_type=jnp.float32)
        + ob_ref[...]
        + xblk
    )


def _fused_kernel(x_ref, wk_ref, wv_ref, a_ref, g_ref, b_ref, ma_ref, mb_ref,
                  ow_ref, ob_ref, out_ref, p_ref):
    i = pl.program_id(0)

    @pl.when(i == 0)
    def _prep():
        x = x_ref[...]
        mu = jnp.mean(x, axis=1, keepdims=True)
        xc = x - mu
        var = jnp.mean(xc * xc, axis=1, keepdims=True)
        h = xc * jax.lax.rsqrt(var + 1e-5) * g_ref[...] + b_ref[...]
        k = jnp.dot(h, wk_ref[...], preferred_element_type=jnp.float32)
        v = jnp.dot(h, wv_ref[...], preferred_element_type=jnp.float32)
        lk = jnp.where(k >= 0, k, _ALPHA * k)
        ska = lk * a_ref[...]
        r = jax.lax.broadcasted_iota(jnp.int32, (_HD, _HD), 0) // _DK
        c = jax.lax.broadcasted_iota(jnp.int32, (_HD, _HD), 1) // _DK
        sel = (r == c).astype(jnp.float32)
        ske = jnp.dot(ska, sel, preferred_element_type=jnp.float32)
        m = jnp.max(ske, axis=0, keepdims=True)
        w = jnp.exp((ske - m) * _SCALE)
        p_ref[:, :_HD] = (w * v).astype(jnp.bfloat16)
        p_ref[:, _HD:] = w.astype(jnp.bfloat16)

    @pl.when((i > 0) & (i % 2 == 1))
    def _agg_even():
        _agg_body(ma_ref, i, p_ref, x_ref, ow_ref, ob_ref, out_ref)

    @pl.when((i > 0) & (i % 2 == 0))
    def _agg_odd():
        _agg_body(mb_ref, i, p_ref, x_ref, ow_ref, ob_ref, out_ref)


def kernel(x, adj_matrix, W_q, W_k, W_v, a, out_W, out_b, ln_gamma, ln_beta):
    del W_q
    wk2 = W_k.transpose(1, 0, 2).reshape(_F, _HD)
    wv2 = W_v.transpose(1, 0, 2).reshape(_F, _HD)
    a2 = a.reshape(1, _HD)
    g2 = ln_gamma.reshape(1, _F)
    b2 = ln_beta.reshape(1, _F)
    ob2 = out_b.reshape(1, _O)

    out = pl.pallas_call(
        _fused_kernel,
        grid=(1 + _N // _BM,),
        in_specs=[
            pl.BlockSpec((_N, _F), lambda i: (0, 0)),
            pl.BlockSpec((_F, _HD), lambda i: (0, 0)),
            pl.BlockSpec((_F, _HD), lambda i: (0, 0)),
            pl.BlockSpec((1, _HD), lambda i: (0, 0)),
            pl.BlockSpec((1, _F), lambda i: (0, 0)),
            pl.BlockSpec((1, _F), lambda i: (0, 0)),
            pl.BlockSpec((_BM, _N),
                         lambda i: (jnp.minimum(2 * (i // 2),
                                                _N // _BM - 2), 0)),
            pl.BlockSpec((_BM, _N),
                         lambda i: (jnp.maximum(2 * ((i - 1) // 2) + 1, 1),
                                    0)),
            pl.BlockSpec((_F, _O), lambda i: (0, 0)),
            pl.BlockSpec((1, _O), lambda i: (0, 0)),
        ],
        out_specs=pl.BlockSpec((_BM, _O), lambda i: (jnp.maximum(i - 1, 0), 0)),
        out_shape=jax.ShapeDtypeStruct((_N, _O), jnp.float32),
        scratch_shapes=[pltpu.VMEM((_N, 2 * _HD), jnp.bfloat16)],
        compiler_params=pltpu.CompilerParams(
            dimension_semantics=("arbitrary",),
        ),
    )(x, wk2, wv2, a2, g2, b2, adj_matrix, adj_matrix, out_W, ob2)
    return out


# speedup vs baseline: 25.4137x; 1.0412x over previous
import math

import jax
import jax.numpy as jnp
from jax.experimental import pallas as pl
from jax.experimental.pallas import tpu as pltpu

_N, _F, _H, _DK, _O = 4096, 128, 4, 32, 128
_ALPHA = 0.2
_SCALE = 1.0 / math.sqrt(_DK)
_HD = _H * _DK
_BM = 512


def _fused_kernel(x_ref, wk_ref, wv_ref, a_ref, g_ref, b_ref, mask_ref,
                  ow_ref, ob_ref, out_ref, p_ref):
    i = pl.program_id(0)

    @pl.when(i == 0)
    def _prep():
        x = x_ref[...]
        mu = jnp.mean(x, axis=1, keepdims=True)
        xc = x - mu
        var = jnp.mean(xc * xc, axis=1, keepdims=True)
        h = xc * jax.lax.rsqrt(var + 1e-5) * g_ref[...] + b_ref[...]
        k = jnp.dot(h, wk_ref[...], preferred_element_type=jnp.float32)
        v = jnp.dot(h, wv_ref[...], preferred_element_type=jnp.float32)
        lk = jnp.where(k >= 0, k, _ALPHA * k)
        ska = lk * a_ref[...]
        r = jax.lax.broadcasted_iota(jnp.int32, (_HD, _HD), 0) // _DK
        c = jax.lax.broadcasted_iota(jnp.int32, (_HD, _HD), 1) // _DK
        sel = (r == c).astype(jnp.float32)
        ske = jnp.dot(ska, sel, preferred_element_type=jnp.float32)
        m = jnp.max(ske, axis=0, keepdims=True)
        w = jnp.exp((ske - m) * _SCALE)
        p_ref[:, :_HD] = (w * v).astype(jnp.bfloat16)
        p_ref[:, _HD:] = w.astype(jnp.bfloat16)

    @pl.when(i > 0)
    def _agg():
        maskf = (mask_ref[...] > 0).astype(jnp.bfloat16)
        agg = jnp.dot(maskf, p_ref[...], preferred_element_type=jnp.float32)
        attn = agg[:, :_HD] / agg[:, _HD:]
        xblk = x_ref[pl.ds((i - 1) * _BM, _BM), :]
        out_ref[...] = (
            jnp.dot(attn, ow_ref[...], preferred_element_type=jnp.float32)
            + ob_ref[...]
            + xblk
        )


def kernel(x, adj_matrix, W_q, W_k, W_v, a, out_W, out_b, ln_gamma, ln_beta):
    del W_q
    wk2 = W_k.transpose(1, 0, 2).reshape(_F, _HD)
    wv2 = W_v.transpose(1, 0, 2).reshape(_F, _HD)
    a2 = a.reshape(1, _HD)
    g2 = ln_gamma.reshape(1, _F)
    b2 = ln_beta.reshape(1, _F)
    ob2 = out_b.reshape(1, _O)

    out = pl.pallas_call(
        _fused_kernel,
        grid=(1 + _N // _BM,),
        in_specs=[
            pl.BlockSpec((_N, _F), lambda i: (0, 0)),
            pl.BlockSpec((_F, _HD), lambda i: (0, 0)),
            pl.BlockSpec((_F, _HD), lambda i: (0, 0)),
            pl.BlockSpec((1, _HD), lambda i: (0, 0)),
            pl.BlockSpec((1, _F), lambda i: (0, 0)),
            pl.BlockSpec((1, _F), lambda i: (0, 0)),
            pl.BlockSpec((_BM, _N),
                         lambda i: (jnp.maximum(i - 1, 0), 0)),
            pl.BlockSpec((_F, _O), lambda i: (0, 0)),
            pl.BlockSpec((1, _O), lambda i: (0, 0)),
        ],
        out_specs=pl.BlockSpec((_BM, _O), lambda i: (jnp.maximum(i - 1, 0), 0)),
        out_shape=jax.ShapeDtypeStruct((_N, _O), jnp.float32),
        scratch_shapes=[pltpu.VMEM((_N, 2 * _HD), jnp.bfloat16)],
        compiler_params=pltpu.CompilerParams(
            dimension_semantics=("arbitrary",),
        ),
    )(x, wk2, wv2, a2, g2, b2, adj_matrix, out_W, ob2)
    return out
